# baseline (speedup 1.0000x reference)
import functools

import jax
import jax.numpy as jnp
from jax import lax
from jax.experimental import pallas as pl
from jax.experimental.pallas import tpu as pltpu
from jax.experimental.pallas import tpu_sc as plsc

PAD_TOKEN_ID = 0

_info = plsc.get_sparse_core_info()
_NC, _NS = _info.num_cores, _info.num_subcores
_NW = _NC * _NS

_S = 4096
_T = 50
_D = 128
_K = 2
_S_PIECE = _S // _K
_S_PER_W = _S_PIECE // _NW
_CHUNK = 2 * _T
_NCHUNK = _S_PER_W // 2
_RING = 8
_LOOK = 6
_NOUTER = _NCHUNK // _RING

_GRP_OFF = (0, 16, 32, 48, 64, 80, 84)


def _make_kernel():
    mesh = plsc.VectorSubcoreMesh(core_axis_name="c", subcore_axis_name="s")

    scratch = [pltpu.VMEM((_NCHUNK, _CHUNK), jnp.int32)]
    scratch += [pltpu.VMEM((_CHUNK, _D), jnp.float32) for _ in range(_RING)]
    scratch += [pltpu.SemaphoreType.DMA for _ in range(2 * _RING)]

    @functools.partial(
        pl.kernel,
        mesh=mesh,
        out_type=jax.ShapeDtypeStruct((_S_PIECE, _T, _D), jnp.float32),
        scratch_types=scratch,
    )
    def emb_kernel(table_hbm, x_hbm, out_hbm, idx_v, *bufs_and_sems):
        bufs = bufs_and_sems[:_RING]
        sems = bufs_and_sems[_RING:2 * _RING]
        ssems = bufs_and_sems[2 * _RING:]
        wid = lax.axis_index("s") * _NC + lax.axis_index("c")
        s0 = wid * _S_PER_W

        pltpu.sync_copy(x_hbm.at[wid], idx_v)

        def gather(j, b):
            pltpu.async_copy(table_hbm.at[idx_v.at[j]], bufs[b], sems[b])

        def wait_gather(j, b):
            pltpu.make_async_copy(
                table_hbm.at[idx_v.at[j]], bufs[b], sems[b]
            ).wait()

        def store(j, b):
            pltpu.async_copy(
                bufs[b].at[pl.ds(0, _T)], out_hbm.at[s0 + 2 * j], ssems[b]
            )
            pltpu.async_copy(
                bufs[b].at[pl.ds(_T, _T)], out_hbm.at[s0 + 2 * j + 1], ssems[b]
            )

        def wait_store(j, b):
            pltpu.make_async_copy(
                bufs[b].at[pl.ds(0, _T)], out_hbm.at[s0 + 2 * j], ssems[b]
            ).wait()
            pltpu.make_async_copy(
                bufs[b].at[pl.ds(_T, _T)], out_hbm.at[s0 + 2 * j + 1], ssems[b]
            ).wait()

        def fix_pads(j, b):
            m = idx_v[j, pl.ds(_GRP_OFF[0], 16)]
            for o in _GRP_OFF[1:]:
                m = jnp.minimum(m, idx_v[j, pl.ds(o, 16)])
            s = m[0]
            for lane in range(1, 16):
                s = jnp.minimum(s, m[lane])

            @pl.when(s == PAD_TOKEN_ID)
            def _fix():
                z = jnp.zeros((16,), jnp.float32)

                def fix_group(g, c2):
                    o = jnp.minimum(g * 16, _CHUNK - 16)
                    vec = idx_v[j, pl.ds(o, 16)]
                    for lane in range(16):
                        @pl.when(vec[lane] == PAD_TOKEN_ID)
                        def _zero(lane=lane):
                            for cb in range(_D // 16):
                                bufs[b][o + lane, pl.ds(cb * 16, 16)] = z
                    return c2

                lax.fori_loop(0, (_CHUNK + 15) // 16, fix_group, 0)

        for jj in range(_LOOK):
            gather(jj, jj)

        def outer(t, carry):
            for b in range(_RING):
                j = t * _RING + b
                wait_gather(j, b)
                fix_pads(j, b)
                store(j, b)

                bb = (b + _LOOK) % _RING

                @pl.when(j + _LOOK < _NCHUNK)
                def _next(j=j, bb=bb):
                    @pl.when(j >= _RING - _LOOK)
                    def _drain():
                        wait_store(j - (_RING - _LOOK), bb)
                    gather(j + _LOOK, bb)
            return carry

        lax.fori_loop(0, _NOUTER, outer, 0)

        for j in range(_NCHUNK - _RING, _NCHUNK):
            wait_store(j, j % _RING)

    return emb_kernel


_emb_kernel = _make_kernel()


@jax.jit
def kernel(embedding, x):
    xi = x.astype(jnp.int32)
    ys = jnp.zeros((_S, _T, _D), jnp.float32)
    for p in range(_K):
        xp = lax.slice(xi, (p * _S_PIECE, 0), ((p + 1) * _S_PIECE, _T))
        xs = xp.reshape(_NW, _NCHUNK, _CHUNK)
        piece = _emb_kernel(embedding, xs)
        ys = lax.dynamic_update_slice(ys, piece, (p * _S_PIECE, 0, 0))
    return ys


# speedup vs baseline: 1.6738x; 1.6738x over previous
import functools

import jax
import jax.numpy as jnp
from jax import lax
from jax.experimental import pallas as pl
from jax.experimental.pallas import tpu as pltpu
from jax.experimental.pallas import tpu_sc as plsc

PAD_TOKEN_ID = 0

_info = plsc.get_sparse_core_info()
_NC, _NS = _info.num_cores, _info.num_subcores
_NW = _NC * _NS

_S = 4096
_T = 50
_D = 128
_S_PER_W = _S // _NW
_CHUNK = 2 * _T
_NCHUNK = _S_PER_W // 2
_RING = 8
_LOOK = 6
_NOUTER = _NCHUNK // _RING

_GRP_OFF = (0, 16, 32, 48, 64, 80, 84)


def _make_kernel():
    mesh = plsc.VectorSubcoreMesh(core_axis_name="c", subcore_axis_name="s")

    scratch = [pltpu.VMEM((_NCHUNK, _CHUNK), jnp.int32)]
    scratch += [pltpu.VMEM((_CHUNK, _D), jnp.float32) for _ in range(_RING)]
    scratch += [pltpu.SemaphoreType.DMA for _ in range(2 * _RING)]

    @functools.partial(
        pl.kernel,
        mesh=mesh,
        out_type=jax.ShapeDtypeStruct((_S, _T, _D), jnp.float32),
        scratch_types=scratch,
    )
    def emb_kernel(table_hbm, x_hbm, out_hbm, idx_v, *bufs_and_sems):
        bufs = bufs_and_sems[:_RING]
        sems = bufs_and_sems[_RING:2 * _RING]
        ssems = bufs_and_sems[2 * _RING:]
        wid = lax.axis_index("s") * _NC + lax.axis_index("c")
        s0 = wid * _S_PER_W

        pltpu.sync_copy(x_hbm.at[wid], idx_v)

        def gather(j, b):
            pltpu.async_copy(table_hbm.at[idx_v.at[j]], bufs[b], sems[b])

        def wait_gather(j, b):
            pltpu.make_async_copy(
                table_hbm.at[idx_v.at[j]], bufs[b], sems[b]
            ).wait()

        def store(j, b):
            pltpu.async_copy(
                bufs[b].at[pl.ds(0, _T)], out_hbm.at[s0 + 2 * j], ssems[b]
            )
            pltpu.async_copy(
                bufs[b].at[pl.ds(_T, _T)], out_hbm.at[s0 + 2 * j + 1], ssems[b]
            )

        def wait_store(j, b):
            pltpu.make_async_copy(
                bufs[b].at[pl.ds(0, _T)], out_hbm.at[s0 + 2 * j], ssems[b]
            ).wait()
            pltpu.make_async_copy(
                bufs[b].at[pl.ds(_T, _T)], out_hbm.at[s0 + 2 * j + 1], ssems[b]
            ).wait()

        def fix_pads(j, b):
            m = idx_v[j, pl.ds(_GRP_OFF[0], 16)]
            for o in _GRP_OFF[1:]:
                m = jnp.minimum(m, idx_v[j, pl.ds(o, 16)])
            s = m[0]
            for lane in range(1, 16):
                s = jnp.minimum(s, m[lane])

            @pl.when(s == PAD_TOKEN_ID)
            def _fix():
                z = jnp.zeros((16,), jnp.float32)

                def fix_group(g, c2):
                    o = jnp.minimum(g * 16, _CHUNK - 16)
                    vec = idx_v[j, pl.ds(o, 16)]
                    for lane in range(16):
                        @pl.when(vec[lane] == PAD_TOKEN_ID)
                        def _zero(lane=lane):
                            for cb in range(_D // 16):
                                bufs[b][o + lane, pl.ds(cb * 16, 16)] = z
                    return c2

                lax.fori_loop(0, (_CHUNK + 15) // 16, fix_group, 0)

        for jj in range(_LOOK):
            gather(jj, jj)

        def outer(t, carry):
            for b in range(_RING):
                j = t * _RING + b
                wait_gather(j, b)
                fix_pads(j, b)
                store(j, b)

                bb = (b + _LOOK) % _RING

                @pl.when(j + _LOOK < _NCHUNK)
                def _next(j=j, bb=bb):
                    @pl.when(j >= _RING - _LOOK)
                    def _drain():
                        wait_store(j - (_RING - _LOOK), bb)
                    gather(j + _LOOK, bb)
            return carry

        lax.fori_loop(0, _NOUTER, outer, 0)

        for j in range(_NCHUNK - _RING, _NCHUNK):
            wait_store(j, j % _RING)

    return emb_kernel


_emb_kernel = _make_kernel()


@jax.jit
def kernel(embedding, x):
    xs = x.reshape(-1).astype(jnp.int32).reshape(_NW, _NCHUNK, _CHUNK)
    return _emb_kernel(embedding, xs)
